# Initial kernel scaffold; baseline (speedup 1.0000x reference)
import jax
import jax.numpy as jnp
from jax.experimental import pallas as pl


def kernel(flat, segment_ids):
    raise NotImplementedError("write your pallas kernel here")


# speedup vs baseline: 5.5995x; 5.5995x over previous
import jax
import jax.numpy as jnp
from jax.experimental import pallas as pl
from jax.experimental.pallas import tpu as pltpu

NUM_SEGMENTS = 16
TOTAL_TOK = 32768
D = 128
BLOCK_TOK = 1024
GRID = TOTAL_TOK // BLOCK_TOK


def _body(ids_ref, flat_ref, out_ref, acc_sum, acc_cnt):
    i = pl.program_id(0)

    @pl.when(i == 0)
    def _zero():
        acc_sum[...] = jnp.zeros_like(acc_sum)
        acc_cnt[...] = jnp.zeros_like(acc_cnt)

    ids = ids_ref[0, 0, :]
    seg_iota = jax.lax.broadcasted_iota(jnp.int32, (BLOCK_TOK, NUM_SEGMENTS), 1)
    onehot = (ids[:, None] == seg_iota).astype(jnp.float32)
    psum = jax.lax.dot_general(
        onehot, flat_ref[...],
        dimension_numbers=(((0,), (0,)), ((), ())),
        preferred_element_type=jnp.float32,
    )
    pcnt = jnp.sum(onehot, axis=0)
    acc_sum[...] += psum
    acc_cnt[...] += jnp.broadcast_to(pcnt[:, None], (NUM_SEGMENTS, D))

    @pl.when(i == GRID - 1)
    def _finish():
        out_ref[...] = acc_sum[...] / jnp.maximum(acc_cnt[...], 1.0)


def kernel(flat, segment_ids):
    ids3 = segment_ids.astype(jnp.int32).reshape(GRID, 1, BLOCK_TOK)
    return pl.pallas_call(
        _body,
        grid=(GRID,),
        in_specs=[
            pl.BlockSpec((1, 1, BLOCK_TOK), lambda i: (i, 0, 0)),
            pl.BlockSpec((BLOCK_TOK, D), lambda i: (i, 0)),
        ],
        out_specs=pl.BlockSpec((NUM_SEGMENTS, D), lambda i: (0, 0)),
        out_shape=jax.ShapeDtypeStruct((NUM_SEGMENTS, D), jnp.float32),
        scratch_shapes=[
            pltpu.VMEM((NUM_SEGMENTS, D), jnp.float32),
            pltpu.VMEM((NUM_SEGMENTS, D), jnp.float32),
        ],
    )(ids3, flat)
